# baseline (speedup 1.0000x reference)
import functools

import jax
import jax.numpy as jnp
from jax.experimental import pallas as pl
from jax.experimental.pallas import tpu as pltpu

_N_QUBITS = 12
_N_LAYERS = 3
_ROWS = 32
_COLS = 128


def _bit_mask(row, col, q):
    if q <= 4:
        return ((row >> (4 - q)) & 1) == 1
    return ((col >> (11 - q)) & 1) == 1


def _partner(x, q, bit):
    if q <= 4:
        sh, ax = 1 << (4 - q), 0
    else:
        sh, ax = 1 << (11 - q), 1
    return jnp.where(bit, jnp.roll(x, sh, axis=ax), jnp.roll(x, -sh, axis=ax))


def _cmul(a, b):
    return (a[0] * b[0] - a[1] * b[1], a[0] * b[1] + a[1] * b[0])


def _apply_1q(re, im, q, bit, U):
    (u00, u01, u10, u11) = U
    pre = _partner(re, q, bit)
    pim = _partner(im, q, bit)
    a0r = jnp.where(bit, pre, re)
    a0i = jnp.where(bit, pim, im)
    a1r = jnp.where(bit, re, pre)
    a1i = jnp.where(bit, im, pim)
    c0r = jnp.where(bit, u10[0], u00[0])
    c0i = jnp.where(bit, u10[1], u00[1])
    c1r = jnp.where(bit, u11[0], u01[0])
    c1i = jnp.where(bit, u11[1], u01[1])
    nre = c0r * a0r - c0i * a0i + c1r * a1r - c1i * a1i
    nim = c0r * a0i + c0i * a0r + c1r * a1i + c1i * a1r
    return nre, nim


def _apply_cnot(re, im, bit_c, t, bit_t):
    pre = _partner(re, t, bit_t)
    pim = _partner(im, t, bit_t)
    return jnp.where(bit_c, pre, re), jnp.where(bit_c, pim, im)


def _qae_kernel(rb, qw_ref, w1t_ref, b1_ref, w2_ref, b2_ref, out_ref, m2_ref):
    @pl.when((pl.program_id(1) == 0) & (pl.program_id(2) == 0))
    def _compute():
        row = jax.lax.broadcasted_iota(jnp.int32, (_ROWS, _COLS), 0)
        col = jax.lax.broadcasted_iota(jnp.int32, (_ROWS, _COLS), 1)
        bits = [_bit_mask(row, col, q) for q in range(_N_QUBITS)]
        re = jnp.where((row + col) == 0, 1.0, 0.0).astype(jnp.float32)
        im = jnp.zeros((_ROWS, _COLS), jnp.float32)

        for l in range(_N_LAYERS):
            for q in range(_N_QUBITS):
                tx = qw_ref[l, 0, q]
                ty = qw_ref[l, 1, q]
                tz = qw_ref[l, 2, q]
                cx, sx = jnp.cos(tx * 0.5), jnp.sin(tx * 0.5)
                cy, sy = jnp.cos(ty * 0.5), jnp.sin(ty * 0.5)
                cz, sz = jnp.cos(tz * 0.5), jnp.sin(tz * 0.5)
                a00 = (cy * cx, sy * sx)
                a01 = (-sy * cx, -cy * sx)
                a10 = (sy * cx, -cy * sx)
                a11 = (cy * cx, -sy * sx)
                em = (cz, -sz)
                ep = (cz, sz)
                U = (_cmul(em, a00), _cmul(em, a01),
                     _cmul(ep, a10), _cmul(ep, a11))
                re, im = _apply_1q(re, im, q, bits[q], U)
            for q in range(_N_QUBITS):
                t = (q + 1) % _N_QUBITS
                re, im = _apply_cnot(re, im, bits[q], t, bits[t])

        probs = re * re + im * im
        acc = b1_ref[...]
        for q in range(_N_QUBITS):
            z_q = jnp.sum(jnp.where(bits[q], -probs, probs))
            acc = acc + z_q * w1t_ref[q:q + 1, :]
        h = jnp.maximum(acc, 0.0)
        pix = jax.nn.sigmoid(
            jax.lax.dot_general(h, w2_ref[...], (((1,), (1,)), ((), ())),
                                preferred_element_type=jnp.float32)
            + b2_ref[...])
        ones_row = jnp.ones((1, _COLS), jnp.float32)
        m = jax.lax.dot_general(pix, ones_row, (((0,), (0,)), ((), ())),
                                preferred_element_type=jnp.float32)
        groups = []
        for d2 in range(4):
            blk = m[4 * d2:4 * d2 + 4, :]
            groups.extend([blk, blk])
        m2_ref[...] = jnp.concatenate(groups, axis=0)

    g = pl.program_id(1)
    tile = m2_ref[pl.ds(g * 8, 8), :].reshape(1, 8, _COLS)
    out_ref[...] = jnp.broadcast_to(tile, (rb, 8, _COLS))


def kernel(images, qweights, W1, b1, W2, b2):
    B = images.shape[0]
    total_rows = B * 16 // _COLS
    rows3 = total_rows // 8
    n_cores = 2
    nb_inner = 1
    rb = rows3 // (4 * n_cores * nb_inner)
    blocks_per_g = n_cores * nb_inner

    out = pl.pallas_call(
        functools.partial(_qae_kernel, rb),
        out_shape=jax.ShapeDtypeStruct((rows3, 8, _COLS), jnp.float32),
        grid=(n_cores, 4, nb_inner),
        in_specs=[
            pl.BlockSpec(memory_space=pltpu.SMEM),
            pl.BlockSpec((12, 32), lambda i, g, j: (0, 0)),
            pl.BlockSpec((1, 32), lambda i, g, j: (0, 0)),
            pl.BlockSpec((16, 32), lambda i, g, j: (0, 0)),
            pl.BlockSpec((1, 16), lambda i, g, j: (0, 0)),
        ],
        out_specs=pl.BlockSpec(
            (rb, 8, _COLS),
            lambda i, g, j, nbg=blocks_per_g, nb=nb_inner:
                (g * nbg + i * nb + j, 0, 0)),
        scratch_shapes=[pltpu.VMEM((32, _COLS), jnp.float32)],
        compiler_params=pltpu.CompilerParams(
            dimension_semantics=("parallel", "arbitrary", "arbitrary"),
            vmem_limit_bytes=48 * 1024 * 1024,
        ),
        name="qae_fused",
    )(qweights.transpose(0, 2, 1), W1.T, b1.reshape(1, 32), W2,
      b2.reshape(1, 16))
    out = out.reshape(4, B // _COLS, 4, _COLS)
    out = out.transpose(1, 3, 0, 2).reshape(B, 1, 4, 4)
    return out


# speedup vs baseline: 1.1120x; 1.1120x over previous
import functools

import jax
import jax.numpy as jnp
from jax.experimental import pallas as pl
from jax.experimental.pallas import tpu as pltpu

_N_QUBITS = 12
_N_LAYERS = 3
_ROWS = 32
_COLS = 128


def _bit_mask(row, col, q):
    if q <= 4:
        return ((row >> (4 - q)) & 1) == 1
    return ((col >> (11 - q)) & 1) == 1


def _partner(x, q, bit):
    if q <= 4:
        sh, ax = 1 << (4 - q), 0
    else:
        sh, ax = 1 << (11 - q), 1
    return jnp.where(bit, jnp.roll(x, sh, axis=ax), jnp.roll(x, -sh, axis=ax))


def _cmul(a, b):
    return (a[0] * b[0] - a[1] * b[1], a[0] * b[1] + a[1] * b[0])


def _apply_1q(re, im, q, bit, U):
    (u00, u01, u10, u11) = U
    pre = _partner(re, q, bit)
    pim = _partner(im, q, bit)
    a0r = jnp.where(bit, pre, re)
    a0i = jnp.where(bit, pim, im)
    a1r = jnp.where(bit, re, pre)
    a1i = jnp.where(bit, im, pim)
    c0r = jnp.where(bit, u10[0], u00[0])
    c0i = jnp.where(bit, u10[1], u00[1])
    c1r = jnp.where(bit, u11[0], u01[0])
    c1i = jnp.where(bit, u11[1], u01[1])
    nre = c0r * a0r - c0i * a0i + c1r * a1r - c1i * a1i
    nim = c0r * a0i + c0i * a0r + c1r * a1i + c1i * a1r
    return nre, nim


def _apply_cnot(re, im, bit_c, t, bit_t):
    pre = _partner(re, t, bit_t)
    pim = _partner(im, t, bit_t)
    return jnp.where(bit_c, pre, re), jnp.where(bit_c, pim, im)


def _qae_kernel(rb, qw_ref, w1t_ref, b1_ref, w2_ref, b2_ref, out_ref, m2_ref):
    @pl.when((pl.program_id(1) == 0) & (pl.program_id(2) == 0))
    def _compute():
        row = jax.lax.broadcasted_iota(jnp.int32, (_ROWS, _COLS), 0)
        col = jax.lax.broadcasted_iota(jnp.int32, (_ROWS, _COLS), 1)
        bits = [_bit_mask(row, col, q) for q in range(_N_QUBITS)]
        re = jnp.where((row + col) == 0, 1.0, 0.0).astype(jnp.float32)
        im = jnp.zeros((_ROWS, _COLS), jnp.float32)

        for l in range(_N_LAYERS):
            for q in range(_N_QUBITS):
                tx = qw_ref[l, 0, q]
                ty = qw_ref[l, 1, q]
                tz = qw_ref[l, 2, q]
                cx, sx = jnp.cos(tx * 0.5), jnp.sin(tx * 0.5)
                cy, sy = jnp.cos(ty * 0.5), jnp.sin(ty * 0.5)
                cz, sz = jnp.cos(tz * 0.5), jnp.sin(tz * 0.5)
                a00 = (cy * cx, sy * sx)
                a01 = (-sy * cx, -cy * sx)
                a10 = (sy * cx, -cy * sx)
                a11 = (cy * cx, -sy * sx)
                em = (cz, -sz)
                ep = (cz, sz)
                U = (_cmul(em, a00), _cmul(em, a01),
                     _cmul(ep, a10), _cmul(ep, a11))
                re, im = _apply_1q(re, im, q, bits[q], U)
            for q in range(_N_QUBITS):
                t = (q + 1) % _N_QUBITS
                re, im = _apply_cnot(re, im, bits[q], t, bits[t])

        probs = re * re + im * im
        acc = b1_ref[...]
        for q in range(_N_QUBITS):
            z_q = jnp.sum(jnp.where(bits[q], -probs, probs))
            acc = acc + z_q * w1t_ref[q:q + 1, :]
        h = jnp.maximum(acc, 0.0)
        pix = jax.nn.sigmoid(
            jax.lax.dot_general(h, w2_ref[...], (((1,), (1,)), ((), ())),
                                preferred_element_type=jnp.float32)
            + b2_ref[...])
        ones_row = jnp.ones((1, _COLS), jnp.float32)
        m = jax.lax.dot_general(pix, ones_row, (((0,), (0,)), ((), ())),
                                preferred_element_type=jnp.float32)
        groups = []
        for d2 in range(4):
            blk = m[4 * d2:4 * d2 + 4, :]
            groups.extend([blk, blk])
        m2_ref[...] = jnp.concatenate(groups, axis=0)

    g = pl.program_id(1)
    tile = m2_ref[pl.ds(g * 8, 8), :].reshape(1, 8, _COLS)
    out_ref[...] = jnp.broadcast_to(tile, (rb, 8, _COLS))


def kernel(images, qweights, W1, b1, W2, b2):
    B = images.shape[0]
    total_rows = B * 16 // _COLS
    rows3 = total_rows // 8
    n_cores = 2
    nb_inner = 1
    rb = rows3 // (4 * n_cores * nb_inner)
    blocks_per_g = n_cores * nb_inner

    out = pl.pallas_call(
        functools.partial(_qae_kernel, rb),
        out_shape=jax.ShapeDtypeStruct((rows3, 8, _COLS), jnp.float32),
        grid=(n_cores, 4, nb_inner),
        in_specs=[
            pl.BlockSpec(memory_space=pltpu.SMEM),
            pl.BlockSpec((12, 32), lambda i, g, j: (0, 0)),
            pl.BlockSpec((1, 32), lambda i, g, j: (0, 0)),
            pl.BlockSpec((16, 32), lambda i, g, j: (0, 0)),
            pl.BlockSpec((1, 16), lambda i, g, j: (0, 0)),
        ],
        out_specs=pl.BlockSpec(
            (rb, 8, _COLS),
            lambda i, g, j, nbg=blocks_per_g, nb=nb_inner:
                (g * nbg + i * nb + j, 0, 0)),
        scratch_shapes=[pltpu.VMEM((32, _COLS), jnp.float32)],
        compiler_params=pltpu.CompilerParams(
            dimension_semantics=("parallel", "arbitrary", "arbitrary"),
        ),
        name="qae_fused",
    )(qweights.transpose(0, 2, 1), W1.T, b1.reshape(1, 32), W2,
      b2.reshape(1, 16))
    out = out.reshape(4, B // _COLS, 4, _COLS)
    out = out.transpose(1, 3, 0, 2).reshape(B, 1, 4, 4)
    return out


# speedup vs baseline: 1.1708x; 1.0528x over previous
import functools

import jax
import jax.numpy as jnp
from jax.experimental import pallas as pl
from jax.experimental.pallas import tpu as pltpu

_N_QUBITS = 12
_N_LAYERS = 3
_ROWS = 32
_COLS = 128


def _bit_mask(row, col, q):
    if q <= 4:
        return ((row >> (4 - q)) & 1) == 1
    return ((col >> (11 - q)) & 1) == 1


def _partner(x, q, bit):
    if q <= 4:
        sh, ax = 1 << (4 - q), 0
    else:
        sh, ax = 1 << (11 - q), 1
    return jnp.where(bit, jnp.roll(x, sh, axis=ax), jnp.roll(x, -sh, axis=ax))


def _cmul(a, b):
    return (a[0] * b[0] - a[1] * b[1], a[0] * b[1] + a[1] * b[0])


def _apply_1q(re, im, q, bit, U):
    (u00, u01, u10, u11) = U
    pre = _partner(re, q, bit)
    pim = _partner(im, q, bit)
    a0r = jnp.where(bit, pre, re)
    a0i = jnp.where(bit, pim, im)
    a1r = jnp.where(bit, re, pre)
    a1i = jnp.where(bit, im, pim)
    c0r = jnp.where(bit, u10[0], u00[0])
    c0i = jnp.where(bit, u10[1], u00[1])
    c1r = jnp.where(bit, u11[0], u01[0])
    c1i = jnp.where(bit, u11[1], u01[1])
    nre = c0r * a0r - c0i * a0i + c1r * a1r - c1i * a1i
    nim = c0r * a0i + c0i * a0r + c1r * a1i + c1i * a1r
    return nre, nim


def _apply_cnot(re, im, bit_c, t, bit_t):
    pre = _partner(re, t, bit_t)
    pim = _partner(im, t, bit_t)
    return jnp.where(bit_c, pre, re), jnp.where(bit_c, pim, im)


def _prefix_xor_perm(n_bits, size):
    a = jax.lax.broadcasted_iota(jnp.int32, (size, size), 0)
    b = jax.lax.broadcasted_iota(jnp.int32, (size, size), 1)
    g = a
    for s in range(1, n_bits):
        g = g ^ (a >> s)
    return (b == g).astype(jnp.float32)


def _qae_kernel(rb, qw_ref, w1t_ref, b1_ref, w2_ref, b2_ref, out_ref, m2_ref):
    @pl.when((pl.program_id(1) == 0) & (pl.program_id(2) == 0))
    def _compute():
        row = jax.lax.broadcasted_iota(jnp.int32, (_ROWS, _COLS), 0)
        col = jax.lax.broadcasted_iota(jnp.int32, (_ROWS, _COLS), 1)
        bits = [_bit_mask(row, col, q) for q in range(_N_QUBITS)]
        re = jnp.where((row + col) == 0, 1.0, 0.0).astype(jnp.float32)
        im = jnp.zeros((_ROWS, _COLS), jnp.float32)

        for l in range(_N_LAYERS):
            for q in range(_N_QUBITS):
                tx = qw_ref[l, 0, q]
                ty = qw_ref[l, 1, q]
                tz = qw_ref[l, 2, q]
                cx, sx = jnp.cos(tx * 0.5), jnp.sin(tx * 0.5)
                cy, sy = jnp.cos(ty * 0.5), jnp.sin(ty * 0.5)
                cz, sz = jnp.cos(tz * 0.5), jnp.sin(tz * 0.5)
                a00 = (cy * cx, sy * sx)
                a01 = (-sy * cx, -cy * sx)
                a10 = (sy * cx, -cy * sx)
                a11 = (cy * cx, -sy * sx)
                em = (cz, -sz)
                ep = (cz, sz)
                U = (_cmul(em, a00), _cmul(em, a01),
                     _cmul(ep, a10), _cmul(ep, a11))
                re, im = _apply_1q(re, im, q, bits[q], U)
            prow = _prefix_xor_perm(5, _ROWS)
            qlane = _prefix_xor_perm(7, _COLS)
            re = jax.lax.dot_general(prow, re, (((0,), (0,)), ((), ())),
                                     preferred_element_type=jnp.float32)
            im = jax.lax.dot_general(prow, im, (((0,), (0,)), ((), ())),
                                     preferred_element_type=jnp.float32)
            pre = jnp.roll(re, 64, axis=1)
            pim = jnp.roll(im, 64, axis=1)
            re = jnp.where(bits[4], pre, re)
            im = jnp.where(bits[4], pim, im)
            re = jnp.dot(re, qlane, preferred_element_type=jnp.float32)
            im = jnp.dot(im, qlane, preferred_element_type=jnp.float32)
            pre = jnp.roll(re, 16, axis=0)
            pim = jnp.roll(im, 16, axis=0)
            re = jnp.where(bits[11], pre, re)
            im = jnp.where(bits[11], pim, im)

        probs = re * re + im * im
        acc = b1_ref[...]
        for q in range(_N_QUBITS):
            z_q = jnp.sum(jnp.where(bits[q], -probs, probs))
            acc = acc + z_q * w1t_ref[q:q + 1, :]
        h = jnp.maximum(acc, 0.0)
        pix = jax.nn.sigmoid(
            jax.lax.dot_general(h, w2_ref[...], (((1,), (1,)), ((), ())),
                                preferred_element_type=jnp.float32)
            + b2_ref[...])
        ones_row = jnp.ones((1, _COLS), jnp.float32)
        m = jax.lax.dot_general(pix, ones_row, (((0,), (0,)), ((), ())),
                                preferred_element_type=jnp.float32)
        groups = []
        for d2 in range(4):
            blk = m[4 * d2:4 * d2 + 4, :]
            groups.extend([blk, blk])
        m2_ref[...] = jnp.concatenate(groups, axis=0)

    g = pl.program_id(1)
    tile = m2_ref[pl.ds(g * 8, 8), :].reshape(1, 8, _COLS)
    out_ref[...] = jnp.broadcast_to(tile, (rb, 8, _COLS))


def kernel(images, qweights, W1, b1, W2, b2):
    B = images.shape[0]
    total_rows = B * 16 // _COLS
    rows3 = total_rows // 8
    n_cores = 2
    nb_inner = 1
    rb = rows3 // (4 * n_cores * nb_inner)
    blocks_per_g = n_cores * nb_inner

    out = pl.pallas_call(
        functools.partial(_qae_kernel, rb),
        out_shape=jax.ShapeDtypeStruct((rows3, 8, _COLS), jnp.float32),
        grid=(n_cores, 4, nb_inner),
        in_specs=[
            pl.BlockSpec(memory_space=pltpu.SMEM),
            pl.BlockSpec((12, 32), lambda i, g, j: (0, 0)),
            pl.BlockSpec((1, 32), lambda i, g, j: (0, 0)),
            pl.BlockSpec((16, 32), lambda i, g, j: (0, 0)),
            pl.BlockSpec((1, 16), lambda i, g, j: (0, 0)),
        ],
        out_specs=pl.BlockSpec(
            (rb, 8, _COLS),
            lambda i, g, j, nbg=blocks_per_g, nb=nb_inner:
                (g * nbg + i * nb + j, 0, 0)),
        scratch_shapes=[pltpu.VMEM((32, _COLS), jnp.float32)],
        compiler_params=pltpu.CompilerParams(
            dimension_semantics=("parallel", "arbitrary", "arbitrary"),
        ),
        name="qae_fused",
    )(qweights.transpose(0, 2, 1), W1.T, b1.reshape(1, 32), W2,
      b2.reshape(1, 16))
    out = out.reshape(4, B // _COLS, 4, _COLS)
    out = out.transpose(1, 3, 0, 2).reshape(B, 1, 4, 4)
    return out
